# Initial kernel scaffold; baseline (speedup 1.0000x reference)
import jax
import jax.numpy as jnp
from jax.experimental import pallas as pl


def kernel(spatial_nodes, sphere_nodes, edge_attr, senders, receivers, W_embed, b_embed, W_e1, b_e1, g_e, o_e, W_e2, b_e2, W_n1, b_n1, g_n, o_n, W_n2, b_n2):
    raise NotImplementedError("write your pallas kernel here")


# speedup vs baseline: 3.1245x; 3.1245x over previous
import functools

import jax
import jax.numpy as jnp
from jax import lax
from jax.experimental import pallas as pl
from jax.experimental.pallas import tpu as pltpu
from jax.experimental.pallas import tpu_sc as plsc

N_SPATIAL = 100000
N_SPHERE = 10000
E = 600000
L = 128
STEPS = 3

NC, NS = 2, 16
NW = NC * NS
IR = 4
CHUNK = IR * 128
IRS = 2
CHUNKS = IRS * 128
EDGE_ROWS = -(-E // 128)
ROWS_PER_W = -(-EDGE_ROWS // (NW * IR)) * IR
N_CHUNKS = ROWS_PER_W // IR
N_CHUNKS_S = ROWS_PER_W // IRS
E_PAD = ROWS_PER_W * NW * 128
HD = L
MSG_ROWS = 10240
STRIPE = MSG_ROWS // NS

TE = 1024
TN = 1000


def _sc_mesh():
    return plsc.VectorSubcoreMesh(
        core_axis_name="c", subcore_axis_name="s",
        num_cores=NC, num_subcores=NS)


def _gather_body(tbl, idx, out, idx_v, rows_v, sem):
    w = lax.axis_index("s") * NC + lax.axis_index("c")

    def body(j, carry):
        r0 = w * ROWS_PER_W + j * IR
        pltpu.sync_copy(idx.at[pl.ds(r0, IR)], idx_v)
        cps = [pltpu.async_copy(tbl.at[idx_v.at[b]],
                                rows_v.at[pl.ds(b * 128, 128)], sem)
               for b in range(IR)]
        for cp in cps:
            cp.wait()
        pltpu.sync_copy(rows_v, out.at[pl.ds(r0 * 128, CHUNK)])
        return carry

    lax.fori_loop(0, N_CHUNKS, body, 0)


def _sc_gather(table, idx2d):
    k = pl.kernel(
        _gather_body,
        out_type=jax.ShapeDtypeStruct((E_PAD, L), jnp.float32),
        mesh=_sc_mesh(),
        scratch_types=[
            pltpu.VMEM((IR, 128), jnp.int32),
            pltpu.VMEM((CHUNK, L), jnp.float32),
            pltpu.SemaphoreType.DMA,
        ],
    )
    return k(table, idx2d)


def _scatter_body(h, idx, zz, out, idx_v, rows_v, acc, sem):
    c = lax.axis_index("c")
    s = lax.axis_index("s")
    w = s * NC + c
    pltpu.sync_copy(zz.at[pl.ds(s * STRIPE, STRIPE)],
                    acc.at[pl.ds(s * STRIPE, STRIPE)])
    plsc.subcore_barrier()

    def body(j, carry):
        r0 = w * ROWS_PER_W + j * IRS
        pltpu.sync_copy(idx.at[pl.ds(r0, IRS)], idx_v)
        pltpu.sync_copy(h.at[pl.ds(r0 * 128, CHUNKS)], rows_v)
        for b in range(IRS):
            pltpu.sync_copy(rows_v.at[pl.ds(b * 128, 128)],
                            acc.at[idx_v.at[b]], add=True)
        return carry

    lax.fori_loop(0, N_CHUNKS_S, body, 0)
    plsc.subcore_barrier()
    pltpu.sync_copy(acc.at[pl.ds(s * STRIPE, STRIPE)],
                    out.at[c, pl.ds(s * STRIPE, STRIPE)])


def _sc_segment_sum(h_ext, rcv2d, zz):
    k = pl.kernel(
        _scatter_body,
        out_type=jax.ShapeDtypeStruct((NC, MSG_ROWS, HD), jnp.float32),
        mesh=_sc_mesh(),
        scratch_types=[
            pltpu.VMEM((IRS, 128), jnp.int32),
            pltpu.VMEM((CHUNKS, HD), jnp.float32),
            pltpu.VMEM_SHARED((MSG_ROWS, HD), jnp.float32),
            pltpu.SemaphoreType.DMA,
        ],
    )
    return k(h_ext, rcv2d, zz)


def _ln(x, g, o):
    m = jnp.mean(x, axis=-1, keepdims=True)
    d = x - m
    v = jnp.mean(d * d, axis=-1, keepdims=True)
    return d * lax.rsqrt(v + 1e-5) * g + o


def _proj_body(x_ref, w_ref, o_ref):
    o_ref[...] = jnp.dot(x_ref[...], w_ref[...],
                         preferred_element_type=jnp.float32)


def _tc_proj(x, w, tile):
    rows = x.shape[0]
    return pl.pallas_call(
        _proj_body,
        grid=(rows // tile,),
        in_specs=[
            pl.BlockSpec((tile, L), lambda i: (i, 0)),
            pl.BlockSpec((L, L), lambda i: (0, 0)),
        ],
        out_specs=pl.BlockSpec((tile, L), lambda i: (i, 0)),
        out_shape=jax.ShapeDtypeStruct((rows, L), jnp.float32),
    )(x, w)


def _edge1_body(ea_ref, sp_ref, rp_ref, wemb_ref, wa_ref, we2_ref, vecs_ref,
                o_ref):
    e0 = jnp.dot(ea_ref[...], wemb_ref[...],
                 preferred_element_type=jnp.float32) + vecs_ref[4:5]
    base = sp_ref[...] + rp_ref[...] + vecs_ref[0:1]
    _edge_tail_from(e0, base, wa_ref, we2_ref, vecs_ref, o_ref)


def _edge23_body(ed_ref, sp_ref, rp_ref, wa_ref, we2_ref, vecs_ref, o_ref):
    base = sp_ref[...] + rp_ref[...] + vecs_ref[0:1]
    _edge_tail_from(ed_ref[...], base, wa_ref, we2_ref, vecs_ref, o_ref)


def _edge_tail_from(edges, base, wa_ref, we2_ref, vecs_ref, o_ref):
    lin = jnp.dot(edges, wa_ref[...],
                  preferred_element_type=jnp.float32) + base
    h = _ln(jax.nn.relu(lin), vecs_ref[1:2], vecs_ref[2:3])
    o_ref[...] = jnp.dot(h, we2_ref[...],
                         preferred_element_type=jnp.float32) + vecs_ref[3:4]


def _tc_edge(first, x, sproj, rproj, ws, vecs):
    kd = x.shape[1]
    wspecs = [pl.BlockSpec((w.shape[0], L), lambda i: (0, 0)) for w in ws]
    return pl.pallas_call(
        _edge1_body if first else _edge23_body,
        grid=(E_PAD // TE,),
        in_specs=[
            pl.BlockSpec((TE, kd), lambda i: (i, 0)),
            pl.BlockSpec((TE, L), lambda i: (i, 0)),
            pl.BlockSpec((TE, L), lambda i: (i, 0)),
            *wspecs,
            pl.BlockSpec((8, L), lambda i: (0, 0)),
        ],
        out_specs=pl.BlockSpec((TE, HD), lambda i: (i, 0)),
        out_shape=jax.ShapeDtypeStruct((E_PAD, HD), jnp.float32),
    )(x, sproj, rproj, *ws, vecs)


def _node_body(msg_ref, sph_ref, wna_ref, w2n_ref, wn2_ref, wc_ref, vecs_ref,
               s_out, rp_out):
    msg_u = msg_ref[0] + msg_ref[1]
    pre = (jnp.dot(sph_ref[...], wna_ref[...],
                   preferred_element_type=jnp.float32)
           + jnp.dot(msg_u, w2n_ref[...], preferred_element_type=jnp.float32)
           + vecs_ref[0:1])
    h2 = _ln(jax.nn.relu(pre), vecs_ref[1:2], vecs_ref[2:3])
    s_new = jnp.dot(h2, wn2_ref[...],
                    preferred_element_type=jnp.float32) + vecs_ref[3:4]
    s_out[...] = s_new
    rp_out[...] = jnp.dot(s_new, wc_ref[...],
                          preferred_element_type=jnp.float32)


def _tc_node(msgp, sphere, wna, w2n, wn2, wc, vecs):
    return pl.pallas_call(
        _node_body,
        grid=(N_SPHERE // TN,),
        in_specs=[
            pl.BlockSpec((NC, TN, HD), lambda i: (0, i, 0)),
            pl.BlockSpec((TN, L), lambda i: (i, 0)),
            pl.BlockSpec((L, L), lambda i: (0, 0)),
            pl.BlockSpec((L, L), lambda i: (0, 0)),
            pl.BlockSpec((L, L), lambda i: (0, 0)),
            pl.BlockSpec((L, L), lambda i: (0, 0)),
            pl.BlockSpec((8, L), lambda i: (0, 0)),
        ],
        out_specs=[
            pl.BlockSpec((TN, L), lambda i: (i, 0)),
            pl.BlockSpec((TN, L), lambda i: (i, 0)),
        ],
        out_shape=[
            jax.ShapeDtypeStruct((N_SPHERE, L), jnp.float32),
            jax.ShapeDtypeStruct((N_SPHERE, L), jnp.float32),
        ],
    )(msgp, sphere, wna, w2n, wn2, wc, vecs)


def _pad_rows(x, n):
    return jnp.pad(x, ((0, n - x.shape[0]),) + ((0, 0),) * (x.ndim - 1))


def _vecs8(*rows):
    v = jnp.stack(rows)
    return jnp.pad(v, ((0, 8 - v.shape[0]), (0, 0)))


def kernel(spatial_nodes, sphere_nodes, edge_attr, senders, receivers,
           W_embed, b_embed, W_e1, b_e1, g_e, o_e, W_e2, b_e2,
           W_n1, b_n1, g_n, o_n, W_n2, b_n2):
    f32 = jnp.float32
    W_a, W_b, W_c = W_e1[:L], W_e1[L:2 * L], W_e1[2 * L:]
    W_n1a, W_n1b = W_n1[:L], W_n1[L:]
    Wemb8 = jnp.pad(W_embed, ((0, 5), (0, 0)))
    vecs_e = _vecs8(b_e1, g_e, o_e, b_e2, b_embed)
    vecs_n = _vecs8(b_n1, g_n, o_n, b_n2)

    snd2d = _pad_rows(senders[:, None], E_PAD).reshape(-1, 128)
    rg2d = _pad_rows(receivers[:, None], E_PAD).reshape(-1, 128)
    rs2d = jnp.pad(receivers[:, None], ((0, E_PAD - E), (0, 0)),
                   constant_values=N_SPHERE).reshape(-1, 128)
    ea8 = jnp.pad(edge_attr, ((0, E_PAD - E), (0, 5)))
    zz = jnp.zeros((MSG_ROWS, HD), f32)

    sp_tab = _tc_proj(spatial_nodes, W_b, 1000)
    sproj = _sc_gather(sp_tab, snd2d)
    rp_tab = _tc_proj(sphere_nodes, W_c, 1000)

    sphere = sphere_nodes
    h_ext = None
    for step in range(STEPS):
        rproj = _sc_gather(rp_tab, rg2d)
        if step == 0:
            h_ext = _tc_edge(True, ea8, sproj, rproj,
                             (Wemb8, W_a, W_e2), vecs_e)
        else:
            h_ext = _tc_edge(False, h_ext, sproj, rproj,
                             (W_a, W_e2), vecs_e)
        msgp = _sc_segment_sum(h_ext, rs2d, zz)
        sphere, rp_tab = _tc_node(msgp, sphere, W_n1a, W_n1b, W_n2, W_c,
                                  vecs_n)
    return sphere


# speedup vs baseline: 5.1889x; 1.6607x over previous
import functools

import jax
import jax.numpy as jnp
from jax import lax
from jax.experimental import pallas as pl
from jax.experimental.pallas import tpu as pltpu
from jax.experimental.pallas import tpu_sc as plsc

N_SPATIAL = 100000
N_SPHERE = 10000
E = 600000
L = 128
STEPS = 3

NC, NS = 2, 16
NW = NC * NS
IR = 3
CHUNK = IR * 128
CHUNKS = 128
HD = L


def _chunking(chunk):
    n_full = E // chunk
    rem = E - n_full * chunk
    return n_full, -(-n_full // NW), n_full % NW, rem // 128, rem % 128


N_FULL_G, MAX_G, TAIL_W_G, TB_G, TT_G = _chunking(CHUNK)
N_FULL_S, MAX_S, TAIL_W_S, TB_S, TT_S = _chunking(CHUNKS)
_JN_G = (MAX_G + 1) // 2
_JN_S = (MAX_S + 1) // 2
MSG_ROWS = 10112
STRIPE = MSG_ROWS // NS

TE = 1024
TN = 1000
E_TILES = -(-E // TE)


def _sc_mesh():
    return plsc.VectorSubcoreMesh(
        core_axis_name="c", subcore_axis_name="s",
        num_cores=NC, num_subcores=NS)


def _gather_body(tbl, idx, out, ia0, ia1, rb0, rb1,
                 si0, si1, sg0, sg1, so0, so1):
    w = lax.axis_index("s") * NC + lax.axis_index("c")
    ia = (ia0, ia1)
    rb = (rb0, rb1)
    sg = (sg0, sg1)
    so = (so0, so1)
    del si0, si1

    def cid(j):
        return j * NW + w

    def e_of(j):
        return pl.multiple_of(cid(j) * CHUNK, CHUNK)

    def start(j, p):
        @pl.when(cid(j) < N_FULL_G)
        def _():
            @pl.when(j >= 2)
            def _():
                pltpu.make_async_copy(rb[p], out.at[pl.ds(0, CHUNK)],
                                      so[p]).wait()
            pltpu.sync_copy(idx.at[pl.ds(e_of(j), CHUNK)], ia[p])
            for b in range(IR):
                pltpu.async_copy(tbl.at[ia[p].at[pl.ds(b * 128, 128)]],
                                 rb[p].at[pl.ds(b * 128, 128)], sg[p])

    def finish(j, q):
        @pl.when((j >= 0) & (cid(j) < N_FULL_G))
        def _():
            pltpu.make_async_copy(out.at[pl.ds(0, CHUNK)], rb[q],
                                  sg[q]).wait()
            pltpu.async_copy(rb[q], out.at[pl.ds(e_of(j), CHUNK)], so[q])

    def body(j2, carry):
        for p in (0, 1):
            j = j2 * 2 + p
            start(j, p)
            finish(j - 1, 1 - p)
        return carry

    lax.fori_loop(0, _JN_G, body, 0)
    finish(2 * _JN_G - 1, 1)
    for p in (0, 1):
        @pl.when(cid(p) < N_FULL_G)
        def _(p=p):
            pltpu.make_async_copy(rb[p], out.at[pl.ds(0, CHUNK)],
                                  so[p]).wait()

    @pl.when(w == TAIL_W_G)
    def _():
        e0 = N_FULL_G * CHUNK
        n = TB_G * 128 + TT_G
        pltpu.sync_copy(idx.at[pl.ds(e0, n)], ia0.at[pl.ds(0, n)])
        cps = [pltpu.async_copy(tbl.at[ia0.at[pl.ds(b * 128, 128)]],
                                rb0.at[pl.ds(b * 128, 128)], sg0)
               for b in range(TB_G)]
        cps.append(pltpu.async_copy(tbl.at[ia0.at[pl.ds(TB_G * 128, TT_G)]],
                                    rb0.at[pl.ds(TB_G * 128, TT_G)], sg0))
        for cp in cps:
            cp.wait()
        pltpu.sync_copy(rb0.at[pl.ds(0, n)], out.at[pl.ds(e0, n)])


def _sc_gather(table, idx1d):
    k = pl.kernel(
        _gather_body,
        out_type=jax.ShapeDtypeStruct((E, L), jnp.float32),
        mesh=_sc_mesh(),
        scratch_types=[
            pltpu.VMEM((CHUNK,), jnp.int32),
            pltpu.VMEM((CHUNK,), jnp.int32),
            pltpu.VMEM((CHUNK, L), jnp.float32),
            pltpu.VMEM((CHUNK, L), jnp.float32),
            pltpu.SemaphoreType.DMA,
            pltpu.SemaphoreType.DMA,
            pltpu.SemaphoreType.DMA,
            pltpu.SemaphoreType.DMA,
            pltpu.SemaphoreType.DMA,
            pltpu.SemaphoreType.DMA,
        ],
    )
    return k(table, idx1d)


def _scatter_body(h, idx, zz, out, ia0, ia1, idx_t, rb0, rb1, acc,
                  si0, si1, sh0, sh1, sa0, sa1):
    cc = lax.axis_index("c")
    s = lax.axis_index("s")
    w = s * NC + cc
    ia = (ia0, ia1)
    rb = (rb0, rb1)
    si = (si0, si1)
    sh = (sh0, sh1)
    sa = (sa0, sa1)

    pltpu.sync_copy(zz.at[pl.ds(s * STRIPE, STRIPE)],
                    acc.at[pl.ds(s * STRIPE, STRIPE)])
    plsc.subcore_barrier()

    def cid(j):
        return j * NW + w

    def e_of(j):
        return pl.multiple_of(cid(j) * CHUNKS, CHUNKS)

    def start(j, p):
        @pl.when(cid(j) < N_FULL_S)
        def _():
            @pl.when(j >= 2)
            def _():
                pltpu.make_async_copy(rb[p], acc.at[pl.ds(0, CHUNKS)],
                                      sa[p]).wait()
            pltpu.async_copy(idx.at[pl.ds(e_of(j), CHUNKS)], ia[p], si[p])
            pltpu.async_copy(h.at[pl.ds(e_of(j), CHUNKS)], rb[p], sh[p])

    def finish(j, q):
        @pl.when((j >= 0) & (cid(j) < N_FULL_S))
        def _():
            pltpu.make_async_copy(idx.at[pl.ds(0, CHUNKS)], ia[q],
                                  si[q]).wait()
            pltpu.make_async_copy(h.at[pl.ds(0, CHUNKS)], rb[q],
                                  sh[q]).wait()
            pltpu.async_copy(rb[q], acc.at[ia[q]], sa[q], add=True)

    def body(j2, carry):
        for p in (0, 1):
            j = j2 * 2 + p
            start(j, p)
            finish(j - 1, 1 - p)
        return carry

    lax.fori_loop(0, _JN_S, body, 0)
    finish(2 * _JN_S - 1, 1)
    for p in (0, 1):
        @pl.when(cid(p) < N_FULL_S)
        def _(p=p):
            pltpu.make_async_copy(rb[p], acc.at[pl.ds(0, CHUNKS)],
                                  sa[p]).wait()

    @pl.when(w == TAIL_W_S)
    def _():
        e0 = N_FULL_S * CHUNKS
        for b in range(TB_S):
            pltpu.sync_copy(idx.at[pl.ds(e0 + b * 128, 128)], ia0)
            pltpu.sync_copy(h.at[pl.ds(e0 + b * 128, 128)],
                            rb0.at[pl.ds(0, 128)])
            pltpu.sync_copy(rb0.at[pl.ds(0, 128)], acc.at[ia0], add=True)
        pltpu.sync_copy(idx.at[pl.ds(e0 + TB_S * 128, TT_S)], idx_t)
        pltpu.sync_copy(h.at[pl.ds(e0 + TB_S * 128, TT_S)],
                        rb0.at[pl.ds(0, TT_S)])
        pltpu.sync_copy(rb0.at[pl.ds(0, TT_S)], acc.at[idx_t], add=True)

    plsc.subcore_barrier()
    pltpu.sync_copy(acc.at[pl.ds(s * STRIPE, STRIPE)],
                    out.at[cc, pl.ds(s * STRIPE, STRIPE)])


def _sc_segment_sum(h_ext, rcv1d, zz):
    k = pl.kernel(
        _scatter_body,
        out_type=jax.ShapeDtypeStruct((NC, MSG_ROWS, HD), jnp.float32),
        mesh=_sc_mesh(),
        scratch_types=[
            pltpu.VMEM((CHUNKS,), jnp.int32),
            pltpu.VMEM((CHUNKS,), jnp.int32),
            pltpu.VMEM((TT_S,), jnp.int32),
            pltpu.VMEM((CHUNKS, HD), jnp.float32),
            pltpu.VMEM((CHUNKS, HD), jnp.float32),
            pltpu.VMEM_SHARED((MSG_ROWS, HD), jnp.float32),
            pltpu.SemaphoreType.DMA,
            pltpu.SemaphoreType.DMA,
            pltpu.SemaphoreType.DMA,
            pltpu.SemaphoreType.DMA,
            pltpu.SemaphoreType.DMA,
            pltpu.SemaphoreType.DMA,
        ],
    )
    return k(h_ext, rcv1d, zz)


def _ln(x, g, o):
    m = jnp.mean(x, axis=-1, keepdims=True)
    d = x - m
    v = jnp.mean(d * d, axis=-1, keepdims=True)
    return d * lax.rsqrt(v + 1e-5) * g + o


def _proj_body(x_ref, w_ref, o_ref):
    o_ref[...] = jnp.dot(x_ref[...], w_ref[...],
                         preferred_element_type=jnp.float32)


def _tc_proj(x, w, tile):
    rows = x.shape[0]
    return pl.pallas_call(
        _proj_body,
        grid=(rows // tile,),
        in_specs=[
            pl.BlockSpec((tile, L), lambda i: (i, 0)),
            pl.BlockSpec((L, L), lambda i: (0, 0)),
        ],
        out_specs=pl.BlockSpec((tile, L), lambda i: (i, 0)),
        out_shape=jax.ShapeDtypeStruct((rows, L), jnp.float32),
    )(x, w)


def _edge1_body(ea_ref, sp_ref, rp_ref, wemb_ref, wa_ref, we2_ref, vecs_ref,
                o_ref):
    e0 = jnp.dot(ea_ref[...], wemb_ref[...],
                 preferred_element_type=jnp.float32) + vecs_ref[4:5]
    base = sp_ref[...] + rp_ref[...] + vecs_ref[0:1]
    _edge_tail_from(e0, base, wa_ref, we2_ref, vecs_ref, o_ref)


def _edge23_body(ed_ref, sp_ref, rp_ref, wa_ref, we2_ref, vecs_ref, o_ref):
    base = sp_ref[...] + rp_ref[...] + vecs_ref[0:1]
    _edge_tail_from(ed_ref[...], base, wa_ref, we2_ref, vecs_ref, o_ref)


def _edge_tail_from(edges, base, wa_ref, we2_ref, vecs_ref, o_ref):
    lin = jnp.dot(edges, wa_ref[...],
                  preferred_element_type=jnp.float32) + base
    h = _ln(jax.nn.relu(lin), vecs_ref[1:2], vecs_ref[2:3])
    o_ref[...] = jnp.dot(h, we2_ref[...],
                         preferred_element_type=jnp.float32) + vecs_ref[3:4]


def _tc_edge(first, x, sproj, rproj, ws, vecs):
    kd = x.shape[1]
    wspecs = [pl.BlockSpec((w.shape[0], L), lambda i: (0, 0)) for w in ws]
    return pl.pallas_call(
        _edge1_body if first else _edge23_body,
        grid=(E_TILES,),
        in_specs=[
            pl.BlockSpec((TE, kd), lambda i: (i, 0)),
            pl.BlockSpec((TE, L), lambda i: (i, 0)),
            pl.BlockSpec((TE, L), lambda i: (i, 0)),
            *wspecs,
            pl.BlockSpec((8, L), lambda i: (0, 0)),
        ],
        out_specs=pl.BlockSpec((TE, HD), lambda i: (i, 0)),
        out_shape=jax.ShapeDtypeStruct((E, HD), jnp.float32),
    )(x, sproj, rproj, *ws, vecs)


def _node_body(msg_ref, sph_ref, wna_ref, w2n_ref, wn2_ref, wc_ref, vecs_ref,
               s_out, rp_out):
    msg_u = msg_ref[0] + msg_ref[1]
    pre = (jnp.dot(sph_ref[...], wna_ref[...],
                   preferred_element_type=jnp.float32)
           + jnp.dot(msg_u, w2n_ref[...], preferred_element_type=jnp.float32)
           + vecs_ref[0:1])
    h2 = _ln(jax.nn.relu(pre), vecs_ref[1:2], vecs_ref[2:3])
    s_new = jnp.dot(h2, wn2_ref[...],
                    preferred_element_type=jnp.float32) + vecs_ref[3:4]
    s_out[...] = s_new
    rp_out[...] = jnp.dot(s_new, wc_ref[...],
                          preferred_element_type=jnp.float32)


def _tc_node(msgp, sphere, wna, w2n, wn2, wc, vecs):
    return pl.pallas_call(
        _node_body,
        grid=(N_SPHERE // TN,),
        in_specs=[
            pl.BlockSpec((NC, TN, HD), lambda i: (0, i, 0)),
            pl.BlockSpec((TN, L), lambda i: (i, 0)),
            pl.BlockSpec((L, L), lambda i: (0, 0)),
            pl.BlockSpec((L, L), lambda i: (0, 0)),
            pl.BlockSpec((L, L), lambda i: (0, 0)),
            pl.BlockSpec((L, L), lambda i: (0, 0)),
            pl.BlockSpec((8, L), lambda i: (0, 0)),
        ],
        out_specs=[
            pl.BlockSpec((TN, L), lambda i: (i, 0)),
            pl.BlockSpec((TN, L), lambda i: (i, 0)),
        ],
        out_shape=[
            jax.ShapeDtypeStruct((N_SPHERE, L), jnp.float32),
            jax.ShapeDtypeStruct((N_SPHERE, L), jnp.float32),
        ],
    )(msgp, sphere, wna, w2n, wn2, wc, vecs)


def _vecs8(*rows):
    v = jnp.stack(rows)
    return jnp.pad(v, ((0, 8 - v.shape[0]), (0, 0)))


def kernel(spatial_nodes, sphere_nodes, edge_attr, senders, receivers,
           W_embed, b_embed, W_e1, b_e1, g_e, o_e, W_e2, b_e2,
           W_n1, b_n1, g_n, o_n, W_n2, b_n2):
    f32 = jnp.float32
    W_a, W_b, W_c = W_e1[:L], W_e1[L:2 * L], W_e1[2 * L:]
    W_n1a, W_n1b = W_n1[:L], W_n1[L:]
    vecs_e = _vecs8(b_e1, g_e, o_e, b_e2, b_embed)
    vecs_n = _vecs8(b_n1, g_n, o_n, b_n2)

    zz = jnp.zeros((MSG_ROWS, HD), f32)

    sp_tab = _tc_proj(spatial_nodes, W_b, 1000)
    sproj = _sc_gather(sp_tab, senders)
    rp_tab = _tc_proj(sphere_nodes, W_c, 1000)

    sphere = sphere_nodes
    h_ext = None
    for step in range(STEPS):
        rproj = _sc_gather(rp_tab, receivers)
        if step == 0:
            h_ext = _tc_edge(True, edge_attr, sproj, rproj,
                             (W_embed, W_a, W_e2), vecs_e)
        else:
            h_ext = _tc_edge(False, h_ext, sproj, rproj,
                             (W_a, W_e2), vecs_e)
        msgp = _sc_segment_sum(h_ext, receivers, zz)
        sphere, rp_tab = _tc_node(msgp, sphere, W_n1a, W_n1b, W_n2, W_c,
                                  vecs_n)
    return sphere


# speedup vs baseline: 5.7209x; 1.1025x over previous
import functools

import jax
import jax.numpy as jnp
from jax import lax
from jax.experimental import pallas as pl
from jax.experimental.pallas import tpu as pltpu
from jax.experimental.pallas import tpu_sc as plsc

N_SPATIAL = 100000
N_SPHERE = 10000
E = 600000
L = 128
STEPS = 3

NC, NS = 2, 16
NW = NC * NS
IR = 3
CHUNK = IR * 128
CHUNKS = 128
HD = L


def _chunking(n, chunk):
    n_full = n // chunk
    rem = n - n_full * chunk
    return n_full, -(-n_full // NW), n_full % NW, rem // 128, rem % 128


EH0 = 299008
EH1 = E - EH0
SPLITS = ((0, EH0), (EH0, EH1))
MSG_ROWS = 10112
STRIPE = MSG_ROWS // NS

TE = 1024
TN = 1000
E_TILES = -(-E // TE)


def _sc_mesh():
    return plsc.VectorSubcoreMesh(
        core_axis_name="c", subcore_axis_name="s",
        num_cores=NC, num_subcores=NS)


def _make_gather_body(off, n_rows):
    n_full, maxc, tail_w, tb, tt = _chunking(n_rows, CHUNK)
    jn = (maxc + 1) // 2

    def gbody(tbl, idx, out, ia0, ia1, rb0, rb1,
              si0, si1, sg0, sg1, so0, so1):
        w = lax.axis_index("s") * NC + lax.axis_index("c")
        ia = (ia0, ia1)
        rb = (rb0, rb1)
        sg = (sg0, sg1)
        so = (so0, so1)
        del si0, si1

        def cid(j):
            return j * NW + w

        def e_loc(j):
            return pl.multiple_of(cid(j) * CHUNK, 128)

        def e_glb(j):
            return pl.multiple_of(off + cid(j) * CHUNK, 128)

        def start(j, p):
            @pl.when(cid(j) < n_full)
            def _():
                @pl.when(j >= 2)
                def _():
                    pltpu.make_async_copy(rb[p], out.at[pl.ds(0, CHUNK)],
                                          so[p]).wait()
                pltpu.sync_copy(idx.at[pl.ds(e_glb(j), CHUNK)], ia[p])
                for b in range(IR):
                    pltpu.async_copy(tbl.at[ia[p].at[pl.ds(b * 128, 128)]],
                                     rb[p].at[pl.ds(b * 128, 128)], sg[p])

        def finish(j, q):
            @pl.when((j >= 0) & (cid(j) < n_full))
            def _():
                pltpu.make_async_copy(out.at[pl.ds(0, CHUNK)], rb[q],
                                      sg[q]).wait()
                pltpu.async_copy(rb[q], out.at[pl.ds(e_loc(j), CHUNK)],
                                 so[q])

        def body(j2, carry):
            for p in (0, 1):
                j = j2 * 2 + p
                start(j, p)
                finish(j - 1, 1 - p)
            return carry

        lax.fori_loop(0, jn, body, 0)
        finish(2 * jn - 1, 1)
        for p in (0, 1):
            @pl.when(cid(p) < n_full)
            def _(p=p):
                pltpu.make_async_copy(rb[p], out.at[pl.ds(0, CHUNK)],
                                      so[p]).wait()

        if tb * 128 + tt:
            @pl.when(w == tail_w)
            def _():
                e0 = n_full * CHUNK
                n = tb * 128 + tt
                pltpu.sync_copy(idx.at[pl.ds(off + e0, n)],
                                ia0.at[pl.ds(0, n)])
                cps = [pltpu.async_copy(
                    tbl.at[ia0.at[pl.ds(b * 128, 128)]],
                    rb0.at[pl.ds(b * 128, 128)], sg0) for b in range(tb)]
                if tt:
                    cps.append(pltpu.async_copy(
                        tbl.at[ia0.at[pl.ds(tb * 128, tt)]],
                        rb0.at[pl.ds(tb * 128, tt)], sg0))
                for cp in cps:
                    cp.wait()
                pltpu.sync_copy(rb0.at[pl.ds(0, n)], out.at[pl.ds(e0, n)])

    return gbody


def _sc_gather(table, idx1d, off, n_rows):
    k = pl.kernel(
        _make_gather_body(off, n_rows),
        out_type=jax.ShapeDtypeStruct((n_rows, L), jnp.float32),
        mesh=_sc_mesh(),
        scratch_types=[
            pltpu.VMEM((CHUNK,), jnp.int32),
            pltpu.VMEM((CHUNK,), jnp.int32),
            pltpu.VMEM((CHUNK, L), jnp.float32),
            pltpu.VMEM((CHUNK, L), jnp.float32),
            pltpu.SemaphoreType.DMA,
            pltpu.SemaphoreType.DMA,
            pltpu.SemaphoreType.DMA,
            pltpu.SemaphoreType.DMA,
            pltpu.SemaphoreType.DMA,
            pltpu.SemaphoreType.DMA,
        ],
    )
    return k(table, idx1d)


def _make_scatter_body(off, n_rows):
    n_full, maxc, tail_w, tb, tt = _chunking(n_rows, CHUNKS)
    jn = (maxc + 1) // 2

    def sbody(h, idx, zz, out, ia0, ia1, idx_t, rb0, rb1, acc,
              si0, si1, sh0, sh1, sa0, sa1):
        cc = lax.axis_index("c")
        s = lax.axis_index("s")
        w = s * NC + cc
        ia = (ia0, ia1)
        rb = (rb0, rb1)
        si = (si0, si1)
        sh = (sh0, sh1)
        sa = (sa0, sa1)

        pltpu.sync_copy(zz.at[pl.ds(s * STRIPE, STRIPE)],
                        acc.at[pl.ds(s * STRIPE, STRIPE)])
        plsc.subcore_barrier()

        def cid(j):
            return j * NW + w

        def e_loc(j):
            return pl.multiple_of(cid(j) * CHUNKS, 128)

        def e_glb(j):
            return pl.multiple_of(off + cid(j) * CHUNKS, 128)

        def start(j, p):
            @pl.when(cid(j) < n_full)
            def _():
                @pl.when(j >= 2)
                def _():
                    pltpu.make_async_copy(rb[p], acc.at[pl.ds(0, CHUNKS)],
                                          sa[p]).wait()
                pltpu.async_copy(idx.at[pl.ds(e_glb(j), CHUNKS)], ia[p],
                                 si[p])
                pltpu.async_copy(h.at[pl.ds(e_loc(j), CHUNKS)], rb[p],
                                 sh[p])

        def finish(j, q):
            @pl.when((j >= 0) & (cid(j) < n_full))
            def _():
                pltpu.make_async_copy(idx.at[pl.ds(0, CHUNKS)], ia[q],
                                      si[q]).wait()
                pltpu.make_async_copy(h.at[pl.ds(0, CHUNKS)], rb[q],
                                      sh[q]).wait()
                pltpu.async_copy(rb[q], acc.at[ia[q]], sa[q], add=True)

        def body(j2, carry):
            for p in (0, 1):
                j = j2 * 2 + p
                start(j, p)
                finish(j - 1, 1 - p)
            return carry

        lax.fori_loop(0, jn, body, 0)
        finish(2 * jn - 1, 1)
        for p in (0, 1):
            @pl.when(cid(p) < n_full)
            def _(p=p):
                pltpu.make_async_copy(rb[p], acc.at[pl.ds(0, CHUNKS)],
                                      sa[p]).wait()

        if tb * 128 + tt:
            @pl.when(w == tail_w)
            def _():
                e0 = n_full * CHUNKS
                for b in range(tb):
                    pltpu.sync_copy(idx.at[pl.ds(off + e0 + b * 128, 128)],
                                    ia0)
                    pltpu.sync_copy(h.at[pl.ds(e0 + b * 128, 128)],
                                    rb0.at[pl.ds(0, 128)])
                    pltpu.sync_copy(rb0.at[pl.ds(0, 128)], acc.at[ia0],
                                    add=True)
                if tt:
                    pltpu.sync_copy(idx.at[pl.ds(off + e0 + tb * 128, tt)],
                                    idx_t)
                    pltpu.sync_copy(h.at[pl.ds(e0 + tb * 128, tt)],
                                    rb0.at[pl.ds(0, tt)])
                    pltpu.sync_copy(rb0.at[pl.ds(0, tt)], acc.at[idx_t],
                                    add=True)

        plsc.subcore_barrier()
        pltpu.sync_copy(acc.at[pl.ds(s * STRIPE, STRIPE)],
                        out.at[cc, pl.ds(s * STRIPE, STRIPE)])

    return sbody


def _sc_segment_sum(h_ext, rcv1d, zz, off, n_rows):
    k = pl.kernel(
        _make_scatter_body(off, n_rows),
        out_type=jax.ShapeDtypeStruct((NC, MSG_ROWS, HD), jnp.float32),
        mesh=_sc_mesh(),
        scratch_types=[
            pltpu.VMEM((CHUNKS,), jnp.int32),
            pltpu.VMEM((CHUNKS,), jnp.int32),
            pltpu.VMEM((64,), jnp.int32),
            pltpu.VMEM((CHUNKS, HD), jnp.float32),
            pltpu.VMEM((CHUNKS, HD), jnp.float32),
            pltpu.VMEM_SHARED((MSG_ROWS, HD), jnp.float32),
            pltpu.SemaphoreType.DMA,
            pltpu.SemaphoreType.DMA,
            pltpu.SemaphoreType.DMA,
            pltpu.SemaphoreType.DMA,
            pltpu.SemaphoreType.DMA,
            pltpu.SemaphoreType.DMA,
        ],
    )
    return k(h_ext, rcv1d, zz)


def _ln(x, g, o):
    m = jnp.mean(x, axis=-1, keepdims=True)
    d = x - m
    v = jnp.mean(d * d, axis=-1, keepdims=True)
    return d * lax.rsqrt(v + 1e-5) * g + o


def _proj_body(x_ref, w_ref, o_ref):
    o_ref[...] = jnp.dot(x_ref[...], w_ref[...],
                         preferred_element_type=jnp.float32)


def _tc_proj(x, w, tile):
    rows = x.shape[0]
    return pl.pallas_call(
        _proj_body,
        grid=(rows // tile,),
        in_specs=[
            pl.BlockSpec((tile, L), lambda i: (i, 0)),
            pl.BlockSpec((L, L), lambda i: (0, 0)),
        ],
        out_specs=pl.BlockSpec((tile, L), lambda i: (i, 0)),
        out_shape=jax.ShapeDtypeStruct((rows, L), jnp.float32),
    )(x, w)


def _edge1_body(ea_ref, sp_ref, rp_ref, wemb_ref, wa_ref, we2_ref, vecs_ref,
                o_ref):
    e0 = jnp.dot(ea_ref[...], wemb_ref[...],
                 preferred_element_type=jnp.float32) + vecs_ref[4:5]
    base = sp_ref[...] + rp_ref[...] + vecs_ref[0:1]
    _edge_tail_from(e0, base, wa_ref, we2_ref, vecs_ref, o_ref)


def _edge23_body(ed_ref, sp_ref, rp_ref, wa_ref, we2_ref, vecs_ref, o_ref):
    base = sp_ref[...] + rp_ref[...] + vecs_ref[0:1]
    _edge_tail_from(ed_ref[...], base, wa_ref, we2_ref, vecs_ref, o_ref)


def _edge_tail_from(edges, base, wa_ref, we2_ref, vecs_ref, o_ref):
    lin = jnp.dot(edges, wa_ref[...],
                  preferred_element_type=jnp.float32) + base
    h = _ln(jax.nn.relu(lin), vecs_ref[1:2], vecs_ref[2:3])
    o_ref[...] = jnp.dot(h, we2_ref[...],
                         preferred_element_type=jnp.float32) + vecs_ref[3:4]


def _tc_edge(first, x, xboff, sproj, rproj, ws, vecs, n_rows):
    kd = x.shape[1]
    wspecs = [pl.BlockSpec((w.shape[0], L), lambda i: (0, 0)) for w in ws]
    return pl.pallas_call(
        _edge1_body if first else _edge23_body,
        grid=(-(-n_rows // TE),),
        in_specs=[
            pl.BlockSpec((TE, kd), lambda i: (i + xboff, 0)),
            pl.BlockSpec((TE, L), lambda i: (i, 0)),
            pl.BlockSpec((TE, L), lambda i: (i, 0)),
            *wspecs,
            pl.BlockSpec((8, L), lambda i: (0, 0)),
        ],
        out_specs=pl.BlockSpec((TE, HD), lambda i: (i, 0)),
        out_shape=jax.ShapeDtypeStruct((n_rows, HD), jnp.float32),
    )(x, sproj, rproj, *ws, vecs)


def _node_body(msg_ref, msg2_ref, sph_ref, wna_ref, w2n_ref, wn2_ref,
               wc_ref, vecs_ref, s_out, rp_out):
    msg_u = (msg_ref[0] + msg_ref[1]) + (msg2_ref[0] + msg2_ref[1])
    pre = (jnp.dot(sph_ref[...], wna_ref[...],
                   preferred_element_type=jnp.float32)
           + jnp.dot(msg_u, w2n_ref[...], preferred_element_type=jnp.float32)
           + vecs_ref[0:1])
    h2 = _ln(jax.nn.relu(pre), vecs_ref[1:2], vecs_ref[2:3])
    s_new = jnp.dot(h2, wn2_ref[...],
                    preferred_element_type=jnp.float32) + vecs_ref[3:4]
    s_out[...] = s_new
    rp_out[...] = jnp.dot(s_new, wc_ref[...],
                          preferred_element_type=jnp.float32)


def _tc_node(msgp, msgp2, sphere, wna, w2n, wn2, wc, vecs):
    return pl.pallas_call(
        _node_body,
        grid=(N_SPHERE // TN,),
        in_specs=[
            pl.BlockSpec((NC, TN, HD), lambda i: (0, i, 0)),
            pl.BlockSpec((NC, TN, HD), lambda i: (0, i, 0)),
            pl.BlockSpec((TN, L), lambda i: (i, 0)),
            pl.BlockSpec((L, L), lambda i: (0, 0)),
            pl.BlockSpec((L, L), lambda i: (0, 0)),
            pl.BlockSpec((L, L), lambda i: (0, 0)),
            pl.BlockSpec((L, L), lambda i: (0, 0)),
            pl.BlockSpec((8, L), lambda i: (0, 0)),
        ],
        out_specs=[
            pl.BlockSpec((TN, L), lambda i: (i, 0)),
            pl.BlockSpec((TN, L), lambda i: (i, 0)),
        ],
        out_shape=[
            jax.ShapeDtypeStruct((N_SPHERE, L), jnp.float32),
            jax.ShapeDtypeStruct((N_SPHERE, L), jnp.float32),
        ],
    )(msgp, msgp2, sphere, wna, w2n, wn2, wc, vecs)


def _vecs8(*rows):
    v = jnp.stack(rows)
    return jnp.pad(v, ((0, 8 - v.shape[0]), (0, 0)))


def kernel(spatial_nodes, sphere_nodes, edge_attr, senders, receivers,
           W_embed, b_embed, W_e1, b_e1, g_e, o_e, W_e2, b_e2,
           W_n1, b_n1, g_n, o_n, W_n2, b_n2):
    f32 = jnp.float32
    W_a, W_b, W_c = W_e1[:L], W_e1[L:2 * L], W_e1[2 * L:]
    W_n1a, W_n1b = W_n1[:L], W_n1[L:]
    vecs_e = _vecs8(b_e1, g_e, o_e, b_e2, b_embed)
    vecs_n = _vecs8(b_n1, g_n, o_n, b_n2)

    zz = jnp.zeros((MSG_ROWS, HD), f32)

    sp_tab = _tc_proj(spatial_nodes, W_b, 1000)
    sproj = [_sc_gather(sp_tab, senders, off, n) for off, n in SPLITS]
    rp_tab = _tc_proj(sphere_nodes, W_c, 1000)

    sphere = sphere_nodes
    h_ext = [None, None]
    for step in range(STEPS):
        msgp = []
        for i, (off, n) in enumerate(SPLITS):
            rproj = _sc_gather(rp_tab, receivers, off, n)
            if step == 0:
                h_ext[i] = _tc_edge(True, edge_attr, off // TE, sproj[i],
                                    rproj, (W_embed, W_a, W_e2), vecs_e, n)
            else:
                h_ext[i] = _tc_edge(False, h_ext[i], 0, sproj[i], rproj,
                                    (W_a, W_e2), vecs_e, n)
            msgp.append(_sc_segment_sum(h_ext[i], receivers, zz, off, n))
        sphere, rp_tab = _tc_node(msgp[0], msgp[1], sphere, W_n1a, W_n1b,
                                  W_n2, W_c, vecs_n)
    return sphere
